# Initial kernel scaffold; baseline (speedup 1.0000x reference)
import jax
import jax.numpy as jnp
from jax.experimental import pallas as pl


def kernel(unique_emb, history_0, history_1, history_2, label_0, label_1, label_2, W_0, W_1, W_2, b_0, b_1, b_2):
    raise NotImplementedError("write your pallas kernel here")


# speedup vs baseline: 4.4829x; 4.4829x over previous
import functools

import jax
import jax.numpy as jnp
from jax import lax
from jax.experimental import pallas as pl
from jax.experimental.pallas import tpu as pltpu
from jax.experimental.pallas import tpu_sc as plsc

EPS = 1e-9


@functools.lru_cache(maxsize=None)
def _make_bag(V, D, B, Lh, n_hist):
  info = plsc.get_sparse_core_info()
  NC, NS, LANES = info.num_cores, info.num_subcores, info.num_lanes
  NW = NC * NS
  assert B % NW == 0 and D % LANES == 0
  b_per_w = B // NW
  n_bags = n_hist * b_per_w
  n_idx = n_bags * Lh
  C0 = min(128, Lh)
  C1 = Lh - C0
  assert Lh % 8 == 0

  mesh = plsc.VectorSubcoreMesh(core_axis_name="c", subcore_axis_name="s")

  @functools.partial(
      pl.kernel,
      mesh=mesh,
      compiler_params=pltpu.CompilerParams(use_tc_tiling_on_sc=False),
      out_type=jax.ShapeDtypeStruct((n_hist * B, D), jnp.float32),
      scratch_types=[
          pltpu.VMEM((n_idx,), jnp.int32),
          pltpu.VMEM((Lh, D), jnp.float32),
          pltpu.VMEM((Lh, D), jnp.float32),
          pltpu.VMEM((n_bags, D), jnp.float32),
          pltpu.SemaphoreType.DMA,
          pltpu.SemaphoreType.DMA,
      ],
  )
  def bag(table, hist, out, idx_v, buf0, buf1, outv, sem0, sem1):
    wid = lax.axis_index("s") * NC + lax.axis_index("c")
    for i in range(n_hist):
      pltpu.sync_copy(
          hist.at[pl.ds((i * B + wid * b_per_w) * Lh, b_per_w * Lh)],
          idx_v.at[pl.ds(i * b_per_w * Lh, b_per_w * Lh)],
      )

    def start(g, buf, sem):
      off = g * Lh
      cps = [pltpu.make_async_copy(
          table.at[idx_v.at[pl.ds(off, C0)]], buf.at[pl.ds(0, C0)], sem)]
      if C1:
        cps.append(pltpu.make_async_copy(
            table.at[idx_v.at[pl.ds(off + C0, C1)]],
            buf.at[pl.ds(C0, C1)], sem))
      return cps

    def accum(g, buf):
      half = D // LANES
      nacc = Lh // 4

      def body(j, carry):
        r = j * 4
        out_c = []
        for h in range(half):
          s = h * LANES
          v = ((buf[r, pl.ds(s, LANES)] + buf[r + 1, pl.ds(s, LANES)]) +
               (buf[r + 2, pl.ds(s, LANES)] + buf[r + 3, pl.ds(s, LANES)]))
          out_c.append(carry[h] + v)
        return tuple(out_c)

      acc = lax.fori_loop(
          0, nacc, body,
          tuple(jnp.zeros((LANES,), jnp.float32) for _ in range(half)))
      for h in range(half):
        outv[g, pl.ds(h * LANES, LANES)] = acc[h]

    for c in start(0, buf0, sem0):
      c.start()
    for c in start(1, buf1, sem1):
      c.start()

    def loop_body(k, _):
      g = k * 2
      for c in start(g, buf0, sem0):
        c.wait()
      accum(g, buf0)

      @pl.when(g + 2 < n_bags)
      def _():
        for c in start(g + 2, buf0, sem0):
          c.start()

      for c in start(g + 1, buf1, sem1):
        c.wait()
      accum(g + 1, buf1)

      @pl.when(g + 3 < n_bags)
      def _():
        for c in start(g + 3, buf1, sem1):
          c.start()

      return 0

    lax.fori_loop(0, n_bags // 2, loop_body, 0)

    for i in range(n_hist):
      pltpu.sync_copy(
          outv.at[pl.ds(i * b_per_w, b_per_w)],
          out.at[pl.ds(i * B + wid * b_per_w, b_per_w), :],
      )

  return bag


def _epilogue_body(n_hist, B, pooled_ref, lbl_ref, w_ref, b_ref, out_ref):
  loss_sum = jnp.float32(0.0)
  correct = jnp.float32(0.0)
  ptp = jnp.float32(0.0); pfp = jnp.float32(0.0); pfn = jnp.float32(0.0)
  ntp = jnp.float32(0.0); nfp = jnp.float32(0.0); nfn = jnp.float32(0.0)
  eps = jnp.float32(EPS)
  for i in range(n_hist):
    pe = pooled_ref[i]
    sq = jnp.sum(pe * pe, axis=1, keepdims=True)
    normed = pe * lax.rsqrt(jnp.maximum(sq, 1e-12))
    logits = jnp.dot(normed, w_ref[i],
                     preferred_element_type=jnp.float32) + b_ref[i]
    p = jnp.clip(jax.nn.sigmoid(logits), eps, 1.0 - eps)
    lbl = lbl_ref[i]
    loss = -lbl * jnp.log(p) - (1.0 - lbl) * jnp.log(1.0 - p)
    loss_sum = loss_sum + jnp.sum(jnp.mean(loss, axis=0))

    pred_pos = p > 0.5
    is_pos = lbl == 1.0
    f32 = lambda x: jnp.asarray(x, jnp.float32)
    correct = correct + jnp.sum(f32(pred_pos == is_pos))
    ptp = ptp + jnp.sum(f32(jnp.logical_and(is_pos, pred_pos)))
    pfp = pfp + jnp.sum(f32(jnp.logical_and(~is_pos, pred_pos)))
    pfn = pfn + jnp.sum(f32(jnp.logical_and(is_pos, ~pred_pos)))

    pred_neg = p < 0.5
    is_neg = lbl == 0.0
    ntp = ntp + jnp.sum(f32(jnp.logical_and(is_neg, pred_neg)))
    nfp = nfp + jnp.sum(f32(jnp.logical_and(~is_neg, pred_neg)))
    nfn = nfn + jnp.sum(f32(jnp.logical_and(is_neg, ~pred_neg)))

  accuracy = correct / jnp.float32(B * 6 * n_hist)
  pos_recall = ptp / jnp.maximum(eps, ptp + pfn)
  pos_precision = ptp / jnp.maximum(eps, ptp + pfp)
  pos_f1 = 2 * pos_recall * pos_precision / jnp.maximum(
      eps, pos_recall + pos_precision)
  neg_recall = ntp / jnp.maximum(eps, ntp + nfn)
  neg_precision = ntp / jnp.maximum(eps, ntp + nfp)
  neg_f1 = 2 * neg_recall * neg_precision / jnp.maximum(
      eps, neg_recall + neg_precision)
  out_ref[0] = loss_sum
  out_ref[1] = (pos_f1 + neg_f1) / 2.0
  out_ref[2] = accuracy


def _epilogue_call(pooled, lbls, Ws, bs):
  n_hist, B, _ = pooled.shape
  return pl.pallas_call(
      functools.partial(_epilogue_body, n_hist, B),
      out_shape=jax.ShapeDtypeStruct((3,), jnp.float32),
      out_specs=pl.BlockSpec(memory_space=pltpu.SMEM),
  )(pooled, lbls, Ws, bs)


def kernel(unique_emb, history_0, history_1, history_2,
           label_0, label_1, label_2,
           W_0, W_1, W_2, b_0, b_1, b_2):
  V, D = unique_emb.shape
  B, Lh = history_0.shape
  hist = jnp.stack([history_0, history_1, history_2]).reshape(3 * B * Lh)
  pooled = _make_bag(V, D, B, Lh, 3)(unique_emb, hist).reshape(3, B, D)
  lbls = jnp.stack([label_0, label_1, label_2])
  Ws = jnp.stack([W_0, W_1, W_2])
  bs = jnp.stack([b_0, b_1, b_2])[:, None, :]
  o = _epilogue_call(pooled, lbls, Ws, bs)
  return (o[0], o[1], o[2])
